# baseline (speedup 1.0000x reference)
import functools

import jax
import jax.numpy as jnp
from jax import lax
from jax.experimental import pallas as pl
from jax.experimental.pallas import tpu as pltpu
from jax.experimental.pallas import tpu_sc as plsc

N = 10000
IN_DIM = 128
HID = 32
HEADS = 8

NTILES = 32
CH = 128
NP2 = 10112
DUMP = N
RPT = NP2 // 16


def _matmul(xin, w, bm):
    m, k = xin.shape
    nout = w.shape[1]

    def body(x_ref, w_ref, o_ref):
        o_ref[...] = jnp.dot(x_ref[...], w_ref[...],
                             preferred_element_type=jnp.float32)

    return pl.pallas_call(
        body,
        grid=(m // bm,),
        in_specs=[pl.BlockSpec((bm, k), lambda i: (i, 0)),
                  pl.BlockSpec((k, nout), lambda i: (0, 0))],
        out_specs=pl.BlockSpec((bm, nout), lambda i: (i, 0)),
        out_shape=jax.ShapeDtypeStruct((m, nout), jnp.float32),
    )(xin, w)


def _edge_pass(h_tab, ta_tab, td_tab, srcp, dstp, zg, z16, gw, lanes,
               with_den, ep):
    pt = ep // NTILES
    nch = pt // CH
    assert nch % 2 == 0
    mesh = plsc.VectorSubcoreMesh(core_axis_name="c", subcore_axis_name="s")

    out_type = [jax.ShapeDtypeStruct((2 * NP2, gw), jnp.float32)]
    if with_den:
        out_type.append(jax.ShapeDtypeStruct((2 * NP2, 16), jnp.float32))

    scratch = [
        [pltpu.VMEM((CH,), jnp.int32)] * 2,
        [pltpu.VMEM((CH,), jnp.int32)] * 2,
        pltpu.VMEM((CH, 16), jnp.float32),
        pltpu.VMEM((CH, 16), jnp.float32),
        pltpu.VMEM((CH, 16), jnp.float32),
        [pltpu.VMEM((CH, gw), jnp.float32)] * 2,
        pltpu.VMEM_SHARED((NP2, gw), jnp.float32),
        pltpu.VMEM_SHARED((NP2, 16), jnp.float32),
        [pltpu.SemaphoreType.DMA] * 2,
        [pltpu.SemaphoreType.DMA] * 2,
        pltpu.SemaphoreType.DMA,
    ]

    def body(h_hbm, ta_hbm, td_hbm, s_hbm, d_hbm, zg_hbm, z16_hbm, *rest):
        if with_den:
            acc_out, den_out = rest[0], rest[1]
            (idx_s, idx_d, tar, tdr, wbuf, rows, acc_sh, den_sh,
             isem, gsem, tsem) = rest[2:]
        else:
            acc_out = rest[0]
            den_out = None
            (idx_s, idx_d, tar, tdr, wbuf, rows, acc_sh, den_sh,
             isem, gsem, tsem) = rest[1:]

        cid = lax.axis_index("c")
        sid = lax.axis_index("s")
        gt = cid * 16 + sid
        r0 = sid * RPT

        pltpu.sync_copy(zg_hbm.at[pl.ds(r0, RPT)], acc_sh.at[pl.ds(r0, RPT)])
        if with_den:
            pltpu.sync_copy(z16_hbm.at[pl.ds(r0, RPT)],
                            den_sh.at[pl.ds(r0, RPT)])
        plsc.subcore_barrier()

        def start_idx(j, b, sync):
            base = (j * NTILES + gt) * CH
            if sync:
                pltpu.sync_copy(s_hbm.at[pl.ds(base, CH)], idx_s[b])
                pltpu.sync_copy(d_hbm.at[pl.ds(base, CH)], idx_d[b])
            else:
                pltpu.async_copy(s_hbm.at[pl.ds(base, CH)], idx_s[b], isem[b])
                pltpu.async_copy(d_hbm.at[pl.ds(base, CH)], idx_d[b], isem[b])

        def wait_idx(b):
            base = gt * pt
            pltpu.make_async_copy(s_hbm.at[pl.ds(base, CH)],
                                  idx_s[b], isem[b]).wait()
            pltpu.make_async_copy(d_hbm.at[pl.ds(base, CH)],
                                  idx_d[b], isem[b]).wait()

        def start_logit_gathers(b):
            pltpu.async_copy(ta_hbm.at[idx_s[b]], tar, tsem)
            pltpu.async_copy(td_hbm.at[idx_d[b]], tdr, tsem)

        def wait_logit_gathers(b):
            pltpu.make_async_copy(ta_hbm.at[idx_s[b]], tar, tsem).wait()
            pltpu.make_async_copy(td_hbm.at[idx_d[b]], tdr, tsem).wait()

        def start_rows_gather(b):
            pltpu.async_copy(h_hbm.at[idx_s[b]], rows[b], gsem[b])

        def wait_rows_gather(b):
            pltpu.make_async_copy(h_hbm.at[idx_s[b]], rows[b],
                                  gsem[b]).wait()

        start_idx(0, 0, True)
        start_logit_gathers(0)
        start_rows_gather(0)
        start_idx(1, 1, False)

        def pair(io, carry):
            for b in range(2):
                j = io * 2 + b

                wait_logit_gathers(b)

                def ebody(e, c):
                    a = tar[e, :] + tdr[e, :]
                    a = jnp.where(a > 0.0, a, a * 0.2)
                    wbuf[e, :] = jnp.exp(a)
                    return c

                lax.fori_loop(0, CH, ebody, 0, unroll=4)

                @pl.when(j + 1 < nch)
                def _():
                    wait_idx(1 - b)
                    start_logit_gathers(1 - b)
                    start_rows_gather(1 - b)

                wait_rows_gather(b)

                def mbody(e, c):
                    w = wbuf[e, :]
                    for v in range(gw // 16):
                        ws = w[lanes[v]]
                        rows[b][e, pl.ds(v * 16, 16)] = (
                            rows[b][e, pl.ds(v * 16, 16)] * ws)
                    return c

                lax.fori_loop(0, CH, mbody, 0, unroll=2)

                if with_den:
                    pltpu.sync_copy(wbuf, den_sh.at[idx_d[b]], add=True)
                pltpu.sync_copy(rows[b], acc_sh.at[idx_d[b]], add=True)

                @pl.when(j + 2 < nch)
                def _():
                    start_idx(j + 2, b, False)
            return carry

        lax.fori_loop(0, nch // 2, pair, 0)

        plsc.subcore_barrier()

        pltpu.sync_copy(acc_sh.at[pl.ds(r0, RPT)],
                        acc_out.at[pl.ds(cid * NP2 + r0, RPT)])
        if with_den:
            pltpu.sync_copy(den_sh.at[pl.ds(r0, RPT)],
                            den_out.at[pl.ds(cid * NP2 + r0, RPT)])

    fn = pl.kernel(body, out_type=out_type, mesh=mesh, scratch_types=scratch,
                   compiler_params=pltpu.CompilerParams(
                       use_tc_tiling_on_sc=False))
    return fn(h_tab, ta_tab, td_tab, srcp, dstp, zg, z16)


def _l2_project(g0a, g0b, g1a, g1b, d0, d1, r1, b1r, w2cat, bm):
    m = g0a.shape[0]
    nout = w2cat.shape[1]

    def body(a_ref, b_ref, c_ref, e_ref, p_ref, q_ref, r_ref, bias_ref,
             w_ref, o_ref):
        den = jnp.dot(p_ref[...] + q_ref[...], r_ref[...],
                      preferred_element_type=jnp.float32) + 1e-16
        h0 = jnp.maximum((a_ref[...] + b_ref[...]) / den[:, :128]
                         + bias_ref[:, :128], 0.0)
        h1 = jnp.maximum((c_ref[...] + e_ref[...]) / den[:, 128:]
                         + bias_ref[:, 128:], 0.0)
        o_ref[...] = (jnp.dot(h0, w_ref[:128, :],
                              preferred_element_type=jnp.float32)
                      + jnp.dot(h1, w_ref[128:, :],
                                preferred_element_type=jnp.float32))

    return pl.pallas_call(
        body,
        grid=(m // bm,),
        in_specs=[pl.BlockSpec((bm, 128), lambda i: (i, 0)),
                  pl.BlockSpec((bm, 128), lambda i: (i, 0)),
                  pl.BlockSpec((bm, 128), lambda i: (i, 0)),
                  pl.BlockSpec((bm, 128), lambda i: (i, 0)),
                  pl.BlockSpec((bm, 16), lambda i: (i, 0)),
                  pl.BlockSpec((bm, 16), lambda i: (i, 0)),
                  pl.BlockSpec((16, 256), lambda i: (0, 0)),
                  pl.BlockSpec((1, 256), lambda i: (0, 0)),
                  pl.BlockSpec((256, nout), lambda i: (0, 0))],
        out_specs=pl.BlockSpec((bm, nout), lambda i: (i, 0)),
        out_shape=jax.ShapeDtypeStruct((m, nout), jnp.float32),
    )(g0a, g0b, g1a, g1b, d0, d1, r1, b1r, w2cat)


def _final(a0, a1, d0, d1, r2, b2r, bm):
    m = a0.shape[0]

    def body(a_ref, b_ref, p_ref, q_ref, r_ref, bias_ref, o_ref):
        den = jnp.dot(p_ref[...] + q_ref[...], r_ref[...],
                      preferred_element_type=jnp.float32) + 1e-16
        o_ref[...] = jnp.maximum((a_ref[...] + b_ref[...]) / den
                                 + bias_ref[...], 0.0)

    return pl.pallas_call(
        body,
        grid=(m // bm,),
        in_specs=[pl.BlockSpec((bm, 32), lambda i: (i, 0)),
                  pl.BlockSpec((bm, 32), lambda i: (i, 0)),
                  pl.BlockSpec((bm, 16), lambda i: (i, 0)),
                  pl.BlockSpec((bm, 16), lambda i: (i, 0)),
                  pl.BlockSpec((16, 32), lambda i: (0, 0)),
                  pl.BlockSpec((1, 32), lambda i: (0, 0))],
        out_specs=pl.BlockSpec((bm, 32), lambda i: (i, 0)),
        out_shape=jax.ShapeDtypeStruct((m, 32), jnp.float32),
    )(a0, a1, d0, d1, r2, b2r)


def _pad_rows(a):
    return jnp.pad(a, ((0, NP2 - N), (0, 0)))


def kernel(x, edge_index, W1, att_src1, att_dst1, b1, W2, att_src2,
           att_dst2, b2):
    e = edge_index.shape[1]
    epr = e + N
    ep = ((epr + 2 * NTILES * CH - 1) // (2 * NTILES * CH)) * (2 * NTILES * CH)

    loop = jnp.arange(N, dtype=jnp.int32)
    padv = jnp.full((ep - epr,), DUMP, dtype=jnp.int32)
    srcp = jnp.concatenate([edge_index[0].astype(jnp.int32), loop, padv])
    dstp = jnp.concatenate([edge_index[1].astype(jnp.int32), loop, padv])

    eye8 = jnp.eye(HEADS, dtype=jnp.float32)
    as1 = (att_src1[0][:, :, None] * eye8[:, None, :]).reshape(HEADS * HID,
                                                               HEADS)
    ad1 = (att_dst1[0][:, :, None] * eye8[:, None, :]).reshape(HEADS * HID,
                                                               HEADS)
    as1 = jnp.pad(as1, ((0, 0), (0, 16 - HEADS)))
    ad1 = jnp.pad(ad1, ((0, 0), (0, 16 - HEADS)))
    w1cat = jnp.concatenate([W1, W1 @ as1, W1 @ ad1], axis=1)

    as2 = jnp.pad(att_src2[0].reshape(HID, 1), ((0, 0), (0, 15)))
    ad2 = jnp.pad(att_dst2[0].reshape(HID, 1), ((0, 0), (0, 15)))
    w2cat = jnp.concatenate([W2, W2 @ as2, W2 @ ad2], axis=1)

    r1 = (jnp.arange(16, dtype=jnp.int32)[:, None]
          == (jnp.arange(256, dtype=jnp.int32) // HID)[None, :]
          ).astype(jnp.float32)
    r2 = jnp.concatenate([jnp.ones((1, 32), jnp.float32),
                          jnp.zeros((15, 32), jnp.float32)])

    zg128 = jnp.zeros((NP2, 128), jnp.float32)
    zg32 = jnp.zeros((NP2, 32), jnp.float32)
    z16 = jnp.zeros((NP2, 16), jnp.float32)

    o1 = _matmul(x, w1cat, 400)
    hg0 = _pad_rows(o1[:, :128])
    hg1 = _pad_rows(o1[:, 128:256])
    ta1 = _pad_rows(o1[:, 256:272])
    td1 = _pad_rows(o1[:, 272:288])

    lanes0 = [v // 2 for v in range(8)]
    lanes1 = [4 + v // 2 for v in range(8)]
    acc0, den1 = _edge_pass(hg0, ta1, td1, srcp, dstp, zg128, z16,
                            128, lanes0, True, ep)
    (acc1,) = _edge_pass(hg1, ta1, td1, srcp, dstp, zg128, z16,
                         128, lanes1, False, ep)

    o2 = _l2_project(acc0[:N], acc0[NP2:NP2 + N],
                     acc1[:N], acc1[NP2:NP2 + N],
                     den1[:N], den1[NP2:NP2 + N],
                     r1, b1.reshape(1, -1), w2cat, 400)
    h2 = _pad_rows(o2[:, :32])
    ta2 = _pad_rows(o2[:, 32:48])
    td2 = _pad_rows(o2[:, 48:64])

    acc2, den2 = _edge_pass(h2, ta2, td2, srcp, dstp, zg32, z16,
                            32, [0, 0], True, ep)

    return _final(acc2[:N], acc2[NP2:NP2 + N],
                  den2[:N], den2[NP2:NP2 + N],
                  r2, b2.reshape(1, -1), 400)


# speedup vs baseline: 1.2165x; 1.2165x over previous
import functools

import jax
import jax.numpy as jnp
from jax import lax
from jax.experimental import pallas as pl
from jax.experimental.pallas import tpu as pltpu
from jax.experimental.pallas import tpu_sc as plsc

N = 10000
IN_DIM = 128
HID = 32
HEADS = 8

NTILES = 32
CH = 128
NP2 = 10112
DUMP = N
RPT = NP2 // 16


def _matmul(xin, w, bm):
    m, k = xin.shape
    nout = w.shape[1]

    def body(x_ref, w_ref, o_ref):
        o_ref[...] = jnp.dot(x_ref[...], w_ref[...],
                             preferred_element_type=jnp.float32)

    return pl.pallas_call(
        body,
        grid=(m // bm,),
        in_specs=[pl.BlockSpec((bm, k), lambda i: (i, 0)),
                  pl.BlockSpec((k, nout), lambda i: (0, 0))],
        out_specs=pl.BlockSpec((bm, nout), lambda i: (i, 0)),
        out_shape=jax.ShapeDtypeStruct((m, nout), jnp.float32),
    )(xin, w)


def _edge_pass(h_tab, ta_tab, td_tab, srcp, dstp, zg, z16, gw, lanes,
               with_den, ep):
    pt = ep // NTILES
    nch = pt // CH
    assert nch % 2 == 0
    mesh = plsc.VectorSubcoreMesh(core_axis_name="c", subcore_axis_name="s")

    out_type = [jax.ShapeDtypeStruct((2 * NP2, gw), jnp.float32)]
    if with_den:
        out_type.append(jax.ShapeDtypeStruct((2 * NP2, 16), jnp.float32))

    scratch = [
        [pltpu.VMEM((CH,), jnp.int32)] * 2,
        [pltpu.VMEM((CH,), jnp.int32)] * 2,
        pltpu.VMEM((CH, 16), jnp.float32),
        pltpu.VMEM((CH, 16), jnp.float32),
        pltpu.VMEM((CH, 16), jnp.float32),
        [pltpu.VMEM((CH, gw), jnp.float32)] * 2,
        pltpu.VMEM_SHARED((NP2, gw), jnp.float32),
        pltpu.VMEM_SHARED((NP2, 16), jnp.float32),
        [pltpu.SemaphoreType.DMA] * 2,
        [pltpu.SemaphoreType.DMA] * 2,
        pltpu.SemaphoreType.DMA,
    ]

    def body(h_hbm, ta_hbm, td_hbm, s_hbm, d_hbm, zg_hbm, z16_hbm, *rest):
        if with_den:
            acc_out, den_out = rest[0], rest[1]
            (idx_s, idx_d, tar, tdr, wbuf, rows, acc_sh, den_sh,
             isem, gsem, tsem) = rest[2:]
        else:
            acc_out = rest[0]
            den_out = None
            (idx_s, idx_d, tar, tdr, wbuf, rows, acc_sh, den_sh,
             isem, gsem, tsem) = rest[1:]

        cid = lax.axis_index("c")
        sid = lax.axis_index("s")
        gt = cid * 16 + sid
        r0 = sid * RPT

        pltpu.sync_copy(zg_hbm.at[pl.ds(r0, RPT)], acc_sh.at[pl.ds(r0, RPT)])
        if with_den:
            pltpu.sync_copy(z16_hbm.at[pl.ds(r0, RPT)],
                            den_sh.at[pl.ds(r0, RPT)])
        plsc.subcore_barrier()

        def start_idx(j, b, sync):
            base = (j * NTILES + gt) * CH
            if sync:
                pltpu.sync_copy(s_hbm.at[pl.ds(base, CH)], idx_s[b])
                pltpu.sync_copy(d_hbm.at[pl.ds(base, CH)], idx_d[b])
            else:
                pltpu.async_copy(s_hbm.at[pl.ds(base, CH)], idx_s[b], isem[b])
                pltpu.async_copy(d_hbm.at[pl.ds(base, CH)], idx_d[b], isem[b])

        def wait_idx(b):
            base = gt * pt
            pltpu.make_async_copy(s_hbm.at[pl.ds(base, CH)],
                                  idx_s[b], isem[b]).wait()
            pltpu.make_async_copy(d_hbm.at[pl.ds(base, CH)],
                                  idx_d[b], isem[b]).wait()

        def start_logit_gathers(b):
            pltpu.async_copy(ta_hbm.at[idx_s[b]], tar, tsem)
            pltpu.async_copy(td_hbm.at[idx_d[b]], tdr, tsem)

        def wait_logit_gathers(b):
            pltpu.make_async_copy(ta_hbm.at[idx_s[b]], tar, tsem).wait()
            pltpu.make_async_copy(td_hbm.at[idx_d[b]], tdr, tsem).wait()

        def start_rows_gather(b):
            pltpu.async_copy(h_hbm.at[idx_s[b]], rows[b], gsem[b])

        def wait_rows_gather(b):
            pltpu.make_async_copy(h_hbm.at[idx_s[b]], rows[b],
                                  gsem[b]).wait()

        start_idx(0, 0, True)
        start_logit_gathers(0)
        start_rows_gather(0)
        start_idx(1, 1, False)

        def pair(io, carry):
            for b in range(2):
                j = io * 2 + b

                wait_logit_gathers(b)

                def ebody(e, c):
                    a = tar[e, :] + tdr[e, :]
                    a = jnp.where(a > 0.0, a, a * 0.2)
                    wbuf[e, :] = jnp.exp(a)
                    return c

                lax.fori_loop(0, CH, ebody, 0)

                @pl.when(j + 1 < nch)
                def _():
                    wait_idx(1 - b)
                    start_logit_gathers(1 - b)
                    start_rows_gather(1 - b)

                wait_rows_gather(b)

                def mbody(e, c):
                    w = wbuf[e, :]
                    for v in range(gw // 16):
                        ws = w[lanes[v]]
                        rows[b][e, pl.ds(v * 16, 16)] = (
                            rows[b][e, pl.ds(v * 16, 16)] * ws)
                    return c

                lax.fori_loop(0, CH, mbody, 0)

                if with_den:
                    pltpu.sync_copy(wbuf, den_sh.at[idx_d[b]], add=True)
                pltpu.sync_copy(rows[b], acc_sh.at[idx_d[b]], add=True)

                @pl.when(j + 2 < nch)
                def _():
                    start_idx(j + 2, b, False)
            return carry

        lax.fori_loop(0, nch // 2, pair, 0)

        plsc.subcore_barrier()

        pltpu.sync_copy(acc_sh.at[pl.ds(r0, RPT)],
                        acc_out.at[pl.ds(cid * NP2 + r0, RPT)])
        if with_den:
            pltpu.sync_copy(den_sh.at[pl.ds(r0, RPT)],
                            den_out.at[pl.ds(cid * NP2 + r0, RPT)])

    fn = pl.kernel(body, out_type=out_type, mesh=mesh, scratch_types=scratch,
                   compiler_params=pltpu.CompilerParams(
                       use_tc_tiling_on_sc=False))
    return fn(h_tab, ta_tab, td_tab, srcp, dstp, zg, z16)


def _l2_project(g0a, g0b, g1a, g1b, d0, d1, r1, b1r, w2cat, bm):
    m = g0a.shape[0]
    nout = w2cat.shape[1]

    def body(a_ref, b_ref, c_ref, e_ref, p_ref, q_ref, r_ref, bias_ref,
             w_ref, o_ref):
        den = jnp.dot(p_ref[...] + q_ref[...], r_ref[...],
                      preferred_element_type=jnp.float32) + 1e-16
        h0 = jnp.maximum((a_ref[...] + b_ref[...]) / den[:, :128]
                         + bias_ref[:, :128], 0.0)
        h1 = jnp.maximum((c_ref[...] + e_ref[...]) / den[:, 128:]
                         + bias_ref[:, 128:], 0.0)
        o_ref[...] = (jnp.dot(h0, w_ref[:128, :],
                              preferred_element_type=jnp.float32)
                      + jnp.dot(h1, w_ref[128:, :],
                                preferred_element_type=jnp.float32))

    return pl.pallas_call(
        body,
        grid=(m // bm,),
        in_specs=[pl.BlockSpec((bm, 128), lambda i: (i, 0)),
                  pl.BlockSpec((bm, 128), lambda i: (i, 0)),
                  pl.BlockSpec((bm, 128), lambda i: (i, 0)),
                  pl.BlockSpec((bm, 128), lambda i: (i, 0)),
                  pl.BlockSpec((bm, 16), lambda i: (i, 0)),
                  pl.BlockSpec((bm, 16), lambda i: (i, 0)),
                  pl.BlockSpec((16, 256), lambda i: (0, 0)),
                  pl.BlockSpec((1, 256), lambda i: (0, 0)),
                  pl.BlockSpec((256, nout), lambda i: (0, 0))],
        out_specs=pl.BlockSpec((bm, nout), lambda i: (i, 0)),
        out_shape=jax.ShapeDtypeStruct((m, nout), jnp.float32),
    )(g0a, g0b, g1a, g1b, d0, d1, r1, b1r, w2cat)


def _final(a0, a1, d0, d1, r2, b2r, bm):
    m = a0.shape[0]

    def body(a_ref, b_ref, p_ref, q_ref, r_ref, bias_ref, o_ref):
        den = jnp.dot(p_ref[...] + q_ref[...], r_ref[...],
                      preferred_element_type=jnp.float32) + 1e-16
        o_ref[...] = jnp.maximum((a_ref[...] + b_ref[...]) / den
                                 + bias_ref[...], 0.0)

    return pl.pallas_call(
        body,
        grid=(m // bm,),
        in_specs=[pl.BlockSpec((bm, 32), lambda i: (i, 0)),
                  pl.BlockSpec((bm, 32), lambda i: (i, 0)),
                  pl.BlockSpec((bm, 16), lambda i: (i, 0)),
                  pl.BlockSpec((bm, 16), lambda i: (i, 0)),
                  pl.BlockSpec((16, 32), lambda i: (0, 0)),
                  pl.BlockSpec((1, 32), lambda i: (0, 0))],
        out_specs=pl.BlockSpec((bm, 32), lambda i: (i, 0)),
        out_shape=jax.ShapeDtypeStruct((m, 32), jnp.float32),
    )(a0, a1, d0, d1, r2, b2r)


def _pad_rows(a):
    return jnp.pad(a, ((0, NP2 - N), (0, 0)))


def kernel(x, edge_index, W1, att_src1, att_dst1, b1, W2, att_src2,
           att_dst2, b2):
    e = edge_index.shape[1]
    epr = e + N
    ep = ((epr + 2 * NTILES * CH - 1) // (2 * NTILES * CH)) * (2 * NTILES * CH)

    loop = jnp.arange(N, dtype=jnp.int32)
    padv = jnp.full((ep - epr,), DUMP, dtype=jnp.int32)
    srcp = jnp.concatenate([edge_index[0].astype(jnp.int32), loop, padv])
    dstp = jnp.concatenate([edge_index[1].astype(jnp.int32), loop, padv])

    eye8 = jnp.eye(HEADS, dtype=jnp.float32)
    as1 = (att_src1[0][:, :, None] * eye8[:, None, :]).reshape(HEADS * HID,
                                                               HEADS)
    ad1 = (att_dst1[0][:, :, None] * eye8[:, None, :]).reshape(HEADS * HID,
                                                               HEADS)
    as1 = jnp.pad(as1, ((0, 0), (0, 16 - HEADS)))
    ad1 = jnp.pad(ad1, ((0, 0), (0, 16 - HEADS)))
    w1cat = jnp.concatenate([W1, W1 @ as1, W1 @ ad1], axis=1)

    as2 = jnp.pad(att_src2[0].reshape(HID, 1), ((0, 0), (0, 15)))
    ad2 = jnp.pad(att_dst2[0].reshape(HID, 1), ((0, 0), (0, 15)))
    w2cat = jnp.concatenate([W2, W2 @ as2, W2 @ ad2], axis=1)

    r1 = (jnp.arange(16, dtype=jnp.int32)[:, None]
          == (jnp.arange(256, dtype=jnp.int32) // HID)[None, :]
          ).astype(jnp.float32)
    r2 = jnp.concatenate([jnp.ones((1, 32), jnp.float32),
                          jnp.zeros((15, 32), jnp.float32)])

    zg128 = jnp.zeros((NP2, 128), jnp.float32)
    zg32 = jnp.zeros((NP2, 32), jnp.float32)
    z16 = jnp.zeros((NP2, 16), jnp.float32)

    o1 = _matmul(x, w1cat, 400)
    hg0 = _pad_rows(o1[:, :128])
    hg1 = _pad_rows(o1[:, 128:256])
    ta1 = _pad_rows(o1[:, 256:272])
    td1 = _pad_rows(o1[:, 272:288])

    lanes0 = [v // 2 for v in range(8)]
    lanes1 = [4 + v // 2 for v in range(8)]
    acc0, den1 = _edge_pass(hg0, ta1, td1, srcp, dstp, zg128, z16,
                            128, lanes0, True, ep)
    (acc1,) = _edge_pass(hg1, ta1, td1, srcp, dstp, zg128, z16,
                         128, lanes1, False, ep)

    o2 = _l2_project(acc0[:N], acc0[NP2:NP2 + N],
                     acc1[:N], acc1[NP2:NP2 + N],
                     den1[:N], den1[NP2:NP2 + N],
                     r1, b1.reshape(1, -1), w2cat, 400)
    h2 = _pad_rows(o2[:, :32])
    ta2 = _pad_rows(o2[:, 32:48])
    td2 = _pad_rows(o2[:, 48:64])

    acc2, den2 = _edge_pass(h2, ta2, td2, srcp, dstp, zg32, z16,
                            32, [0, 0], True, ep)

    return _final(acc2[:N], acc2[NP2:NP2 + N],
                  den2[:N], den2[NP2:NP2 + N],
                  r2, b2.reshape(1, -1), 400)
